# Initial kernel scaffold; baseline (speedup 1.0000x reference)
import jax
import jax.numpy as jnp
from jax.experimental import pallas as pl


def kernel(V, E, edge_index, u, z, eu_W1, eu_b1, eu_W2, eu_b2, nu_W1, nu_b1, nu_W2, nu_b2, gn_W1, gn_b1, gn_W2, gn_b2, gu_W1, gu_b1, gu_W2, gu_b2):
    raise NotImplementedError("write your pallas kernel here")


# speedup vs baseline: 1.2053x; 1.2053x over previous
import functools

import jax
import jax.numpy as jnp
from jax import lax
from jax.experimental import pallas as pl
from jax.experimental.pallas import tpu as pltpu
from jax.experimental.pallas import tpu_sc as plsc

N = 10000
NE = 320000
D = 128
H = 256
HH = 128
NSUB = 16
EPT = NE // NSUB
C = 40
NCH = EPT // C
NROWCH = N // C
ROWIT = -(-NROWCH // NSUB)

_f32 = jnp.float32


def _sc_edge_aggregate(a_src0, a_src1, a_dst0, a_dst1, b0, b1, src_idx,
                       dst_idx):
  mesh = plsc.VectorSubcoreMesh(core_axis_name="c", subcore_axis_name="s")

  @functools.partial(
      pl.kernel,
      mesh=mesh,
      out_type=(
          jax.ShapeDtypeStruct((N, HH), _f32),
          jax.ShapeDtypeStruct((N, HH), _f32),
      ),
      scratch_types=[
          pltpu.VMEM((C,), jnp.int32),
          pltpu.VMEM((C,), jnp.int32),
          pltpu.VMEM((C, HH), _f32),
          pltpu.VMEM((C, HH), _f32),
          pltpu.VMEM((C, HH), _f32),
          pltpu.VMEM((C, HH), _f32),
          pltpu.VMEM_SHARED((N, HH), _f32),
      ],
  )
  def k(as0_hbm, as1_hbm, ad0_hbm, ad1_hbm, b0_hbm, b1_hbm, src_hbm, dst_hbm,
        t0_hbm, t1_hbm,
        sidx, didx, srows, drows, brows, zrows, acc):
    c = lax.axis_index("c")
    s = lax.axis_index("s")

    zv = jnp.zeros((1, 16), _f32)

    @pl.loop(0, C)
    def _(r):
      for j in range(HH // 16):
        zrows[pl.ds(r, 1), pl.ds(j * 16, 16)] = zv

    for kk in range(ROWIT):
      g = kk * NSUB + s

      @pl.when(g < NROWCH)
      def _():
        pltpu.sync_copy(zrows, acc.at[pl.ds(g * C, C)])

    plsc.subcore_barrier()

    ebase = s * EPT

    @pl.loop(0, NCH)
    def _(i):
      base = ebase + i * C
      pltpu.sync_copy(src_hbm.at[pl.ds(base, C)], sidx)
      pltpu.sync_copy(dst_hbm.at[pl.ds(base, C)], didx)

      @pl.when(c == 0)
      def _():
        pltpu.sync_copy(as0_hbm.at[sidx], srows)
        pltpu.sync_copy(ad0_hbm.at[didx], drows)
        pltpu.sync_copy(b0_hbm.at[pl.ds(base, C)], brows)

      @pl.when(c == 1)
      def _():
        pltpu.sync_copy(as1_hbm.at[sidx], srows)
        pltpu.sync_copy(ad1_hbm.at[didx], drows)
        pltpu.sync_copy(b1_hbm.at[pl.ds(base, C)], brows)

      @pl.loop(0, C)
      def _(r):
        for j in range(HH // 16):
          slr = (pl.ds(r, 1), pl.ds(j * 16, 16))
          brows[slr] = jnp.maximum(brows[slr] + srows[slr] + drows[slr], 0.0)

      pltpu.sync_copy(brows, acc.at[didx], add=True)

    plsc.subcore_barrier()

    for kk in range(ROWIT):
      g = kk * NSUB + s

      @pl.when(g < NROWCH)
      def _():
        off = g * C

        @pl.when(c == 0)
        def _():
          pltpu.sync_copy(acc.at[pl.ds(off, C)], t0_hbm.at[pl.ds(off, C)])

        @pl.when(c == 1)
        def _():
          pltpu.sync_copy(acc.at[pl.ds(off, C)], t1_hbm.at[pl.ds(off, C)])

  return k(a_src0, a_src1, a_dst0, a_dst1, b0, b1, src_idx, dst_idx)


C2 = 80
EPT2 = NE // 2 // NSUB
NCH2 = EPT2 // C2


def _sc_degree(dst_idx):
  mesh = plsc.VectorSubcoreMesh(core_axis_name="c", subcore_axis_name="s")

  @functools.partial(
      pl.kernel,
      mesh=mesh,
      out_type=jax.ShapeDtypeStruct((2, N, HH), _f32),
      scratch_types=[
          pltpu.VMEM((C2,), jnp.int32),
          pltpu.VMEM((C2, HH), _f32),
          pltpu.VMEM((C, HH), _f32),
          pltpu.VMEM_SHARED((N, HH), _f32),
      ],
  )
  def k(dst_hbm, deg_hbm, didx, ones_v, zrows, acc):
    c = lax.axis_index("c")
    s = lax.axis_index("s")

    zv = jnp.zeros((1, 16), _f32)
    ov = jnp.ones((1, 16), _f32)

    @pl.loop(0, C2)
    def _(r):
      for j in range(HH // 16):
        ones_v[pl.ds(r, 1), pl.ds(j * 16, 16)] = ov

    @pl.loop(0, C)
    def _(r):
      for j in range(HH // 16):
        zrows[pl.ds(r, 1), pl.ds(j * 16, 16)] = zv

    for kk in range(ROWIT):
      g = kk * NSUB + s

      @pl.when(g < NROWCH)
      def _():
        pltpu.sync_copy(zrows, acc.at[pl.ds(g * C, C)])

    plsc.subcore_barrier()

    ebase = (c * NSUB + s) * EPT2

    @pl.loop(0, NCH2)
    def _(i):
      pltpu.sync_copy(dst_hbm.at[pl.ds(ebase + i * C2, C2)], didx)
      pltpu.sync_copy(ones_v, acc.at[didx], add=True)

    plsc.subcore_barrier()

    for kk in range(ROWIT):
      g = kk * NSUB + s

      @pl.when(g < NROWCH)
      def _():
        off = g * C
        pltpu.sync_copy(acc.at[pl.ds(off, C)], deg_hbm.at[c].at[pl.ds(off, C)])

  return k(dst_idx)


def _tc_prep(V, Ws, Wd, eu_W2, Wn1_a, eu_b2r, zr, ur, Wn1_z, Wn1_u, nu_b1r):
  def body(v_r, ws_r, wd_r, w2_r, wa_r, b2_r, z_r, u_r, wz_r, wu_r, nb1_r,
           as0_o, as1_o, ad0_o, ad1_o, m_o, crow_o, crow2_o):
    v = v_r[...]
    as0_o[...] = jnp.dot(v, ws_r[0], preferred_element_type=_f32)
    as1_o[...] = jnp.dot(v, ws_r[1], preferred_element_type=_f32)
    ad0_o[...] = jnp.dot(v, wd_r[0], preferred_element_type=_f32)
    ad1_o[...] = jnp.dot(v, wd_r[1], preferred_element_type=_f32)
    m_o[...] = jnp.dot(w2_r[...], wa_r[...], preferred_element_type=_f32)
    crow_o[...] = (jnp.dot(z_r[...], wz_r[...], preferred_element_type=_f32)
                   + jnp.dot(u_r[...], wu_r[...], preferred_element_type=_f32)
                   + nb1_r[...])
    crow2_o[...] = jnp.dot(b2_r[...], wa_r[...], preferred_element_type=_f32)

  nhh = jax.ShapeDtypeStruct((N, HH), _f32)
  return pl.pallas_call(
      body,
      out_shape=[nhh, nhh, nhh, nhh,
                 jax.ShapeDtypeStruct((H, H), _f32),
                 jax.ShapeDtypeStruct((1, H), _f32),
                 jax.ShapeDtypeStruct((1, H), _f32)],
  )(V, Ws, Wd, eu_W2, Wn1_a, eu_b2r, zr, ur, Wn1_z, Wn1_u, nu_b1r)


_BBLK = 16000


def _tc_edge_bias(E, WE, b1r):
  def body(e_r, we_r, b1_r, b0_o, b1_o):
    e = e_r[...]
    b0_o[...] = jnp.dot(e, we_r[0], preferred_element_type=_f32) + b1_r[0]
    b1_o[...] = jnp.dot(e, we_r[1], preferred_element_type=_f32) + b1_r[1]

  nblk = NE // _BBLK
  return pl.pallas_call(
      body,
      grid=(nblk,),
      in_specs=[
          pl.BlockSpec((_BBLK, 16), lambda i: (i, 0)),
          pl.BlockSpec((2, 16, HH), lambda i: (0, 0, 0)),
          pl.BlockSpec((2, 1, HH), lambda i: (0, 0, 0)),
      ],
      out_specs=[
          pl.BlockSpec((_BBLK, HH), lambda i: (i, 0)),
          pl.BlockSpec((_BBLK, HH), lambda i: (i, 0)),
      ],
      out_shape=[jax.ShapeDtypeStruct((NE, HH), _f32),
                 jax.ShapeDtypeStruct((NE, HH), _f32)],
  )(E, WE, b1r)


_NBLK = 1000


def _tc_node(T0, T1, Degs, V, M, Wn1_v, crow, crow2, nu_W2, nu_b2r):
  def body(t0_r, t1_r, dg0_r, dg1_r, v_r, m_r, wv_r, crow_r, crow2_r,
           w2_r, b2_r, vp_o, vsum_o, vmax_o):
    i = pl.program_id(0)
    degraw = dg0_r[:, 0:1] + dg1_r[:, 0:1]
    deg = jnp.maximum(degraw, 1.0)
    ind = (degraw > 0.0).astype(_f32)
    x0 = t0_r[...] / deg
    x1 = t1_r[...] / deg
    pre = (jnp.dot(x0, m_r[0:HH], preferred_element_type=_f32)
           + jnp.dot(x1, m_r[HH:], preferred_element_type=_f32)
           + jnp.dot(v_r[...], wv_r[...], preferred_element_type=_f32)
           + crow_r[...] + ind * crow2_r[...])
    h = jnp.maximum(pre, 0.0)
    out = jnp.dot(h, w2_r[...], preferred_element_type=_f32) + b2_r[...]
    vp_o[...] = out
    psum = jnp.sum(out, axis=0, keepdims=True)
    pmax = jnp.max(out, axis=0, keepdims=True)

    @pl.when(i == 0)
    def _():
      vsum_o[...] = psum
      vmax_o[...] = pmax

    @pl.when(i > 0)
    def _():
      vsum_o[...] = vsum_o[...] + psum
      vmax_o[...] = jnp.maximum(vmax_o[...], pmax)

  full2 = lambda shape: pl.BlockSpec(shape, lambda i: (0, 0))
  return pl.pallas_call(
      body,
      grid=(N // _NBLK,),
      in_specs=[
          pl.BlockSpec((_NBLK, HH), lambda i: (i, 0)),
          pl.BlockSpec((_NBLK, HH), lambda i: (i, 0)),
          pl.BlockSpec((_NBLK, HH), lambda i: (i, 0)),
          pl.BlockSpec((_NBLK, HH), lambda i: (N // _NBLK + i, 0)),
          pl.BlockSpec((_NBLK, D), lambda i: (i, 0)),
          full2((H, H)),
          full2((D, H)),
          full2((1, H)),
          full2((1, H)),
          full2((H, H)),
          full2((1, H)),
      ],
      out_specs=[
          pl.BlockSpec((_NBLK, H), lambda i: (i, 0)),
          full2((1, H)),
          full2((1, H)),
      ],
      out_shape=[
          jax.ShapeDtypeStruct((N, H), _f32),
          jax.ShapeDtypeStruct((1, H), _f32),
          jax.ShapeDtypeStruct((1, H), _f32),
      ],
  )(T0, T1, Degs, Degs, V, M, Wn1_v, crow, crow2, nu_W2, nu_b2r)


def _tc_global(vsum, vmax, zr, ur, gn_W1, gn_b1r, gn_W2, gn_b2r,
               gu_W1, gu_b1r, gu_W2, gu_b2r):
  def body(vs_r, vm_r, z_r, u_r, w1_r, b1_r, w2_r, b2_r,
           uw1_r, ub1_r, uw2_r, ub2_r, zp_o, up_o):
    vmean = vs_r[...] * (1.0 / N)
    h = jnp.maximum(
        jnp.dot(vmean, w1_r[0:H], preferred_element_type=_f32)
        + jnp.dot(z_r[...], w1_r[H:], preferred_element_type=_f32)
        + b1_r[...], 0.0)
    zp = jnp.dot(h, w2_r[...], preferred_element_type=_f32) + b2_r[...]
    zp_o[...] = zp
    h2 = jnp.maximum(
        jnp.dot(vmean, uw1_r[0:H], preferred_element_type=_f32)
        + jnp.dot(vm_r[...], uw1_r[H:2 * H], preferred_element_type=_f32)
        + jnp.dot(zp, uw1_r[2 * H:3 * H], preferred_element_type=_f32)
        + jnp.dot(u_r[...], uw1_r[3 * H:], preferred_element_type=_f32)
        + ub1_r[...], 0.0)
    up_o[...] = jnp.dot(h2, uw2_r[...], preferred_element_type=_f32) + ub2_r[...]

  return pl.pallas_call(
      body,
      out_shape=[
          jax.ShapeDtypeStruct((1, H), _f32),
          jax.ShapeDtypeStruct((1, D), _f32),
      ],
  )(vsum, vmax, zr, ur, gn_W1, gn_b1r, gn_W2, gn_b2r,
    gu_W1, gu_b1r, gu_W2, gu_b2r)


def kernel(V, E, edge_index, u, z,
           eu_W1, eu_b1, eu_W2, eu_b2,
           nu_W1, nu_b1, nu_W2, nu_b2,
           gn_W1, gn_b1, gn_W2, gn_b2,
           gu_W1, gu_b1, gu_W2, gu_b2):
  src = edge_index[0].astype(jnp.int32)
  dst = edge_index[1].astype(jnp.int32)

  WE = eu_W1[:16].reshape(16, 2, HH).transpose(1, 0, 2)
  Ws = eu_W1[16:16 + D].reshape(D, 2, HH).transpose(1, 0, 2)
  Wd = eu_W1[16 + D:].reshape(D, 2, HH).transpose(1, 0, 2)
  b1r = eu_b1.reshape(2, 1, HH)

  Wn1_a = nu_W1[0:H]
  Wn1_v = nu_W1[H:H + D]
  Wn1_z = nu_W1[H + D:2 * H + D]
  Wn1_u = nu_W1[2 * H + D:]

  zr = z.reshape(1, H)
  ur = u.reshape(1, D)

  as0, as1, ad0, ad1, M, crow, crow2 = _tc_prep(
      V, Ws, Wd, eu_W2, Wn1_a, eu_b2.reshape(1, H), zr, ur,
      Wn1_z, Wn1_u, nu_b1.reshape(1, H))
  B0, B1 = _tc_edge_bias(E, WE, b1r)
  Degs = _sc_degree(dst).reshape(2 * N, HH)
  T0, T1 = _sc_edge_aggregate(as0, as1, ad0, ad1, B0, B1, src, dst)
  V_prime, vsum, vmax = _tc_node(T0, T1, Degs, V, M, Wn1_v, crow, crow2,
                                 nu_W2, nu_b2.reshape(1, H))
  zp, up = _tc_global(vsum, vmax, zr, ur,
                      gn_W1, gn_b1.reshape(1, H), gn_W2, gn_b2.reshape(1, H),
                      gu_W1, gu_b1.reshape(1, H), gu_W2, gu_b2.reshape(1, D))
  return (V_prime, up.reshape(D), zp.reshape(H))


# speedup vs baseline: 2.8370x; 2.3538x over previous
import functools

import jax
import jax.numpy as jnp
from jax import lax
from jax.experimental import pallas as pl
from jax.experimental.pallas import tpu as pltpu
from jax.experimental.pallas import tpu_sc as plsc

N = 10000
NE = 320000
D = 128
H = 256
HH = 128
NSUB = 16
EPT = NE // NSUB
C = 40
NCH = EPT // C
NQ = NCH // 4
NROWCH = N // C
ROWIT = -(-NROWCH // NSUB)

_f32 = jnp.float32


def _sc_edge_aggregate(a_src, a_dst, b_edges, src_idx, dst_idx):
  mesh = plsc.VectorSubcoreMesh(core_axis_name="c", subcore_axis_name="s")

  @functools.partial(
      pl.kernel,
      mesh=mesh,
      out_type=jax.ShapeDtypeStruct((2, N, HH), _f32),
      scratch_types=[
          pltpu.VMEM((C,), jnp.int32),
          pltpu.VMEM((C,), jnp.int32),
          pltpu.VMEM((C,), jnp.int32),
          pltpu.VMEM((C,), jnp.int32),
          pltpu.VMEM((C,), jnp.int32),
          pltpu.VMEM((C,), jnp.int32),
          pltpu.VMEM((C,), jnp.int32),
          pltpu.VMEM((C,), jnp.int32),
          pltpu.VMEM((C, HH), _f32),
          pltpu.VMEM((C, HH), _f32),
          pltpu.VMEM((C, HH), _f32),
          pltpu.VMEM((C, HH), _f32),
          pltpu.VMEM((C, HH), _f32),
          pltpu.VMEM((C, HH), _f32),
          pltpu.SemaphoreType.DMA,
          pltpu.SemaphoreType.DMA,
          pltpu.SemaphoreType.DMA,
          pltpu.SemaphoreType.DMA,
          pltpu.SemaphoreType.DMA,
          pltpu.SemaphoreType.DMA,
          pltpu.SemaphoreType.DMA,
          pltpu.SemaphoreType.DMA,
          pltpu.VMEM_SHARED((N, HH), _f32),
      ],
  )
  def k(as_hbm, ad_hbm, b_hbm, src_hbm, dst_hbm, t_hbm,
        si0, si1, si2, si3, di0, di1, di2, di3,
        sr0, sr1, dr0, dr1, br0, br1,
        semg0, semg1, semi0, semi1, semi2, semi3, sems0, sems1, acc):
    c = lax.axis_index("c")
    s = lax.axis_index("s")
    sidx = (si0, si1, si2, si3)
    didx = (di0, di1, di2, di3)
    sr = (sr0, sr1)
    dr = (dr0, dr1)
    br = (br0, br1)
    semg = (semg0, semg1)
    semi = (semi0, semi1, semi2, semi3)
    sems = (sems0, sems1)
    ebase = s * EPT

    zv = jnp.zeros((1, 16), _f32)

    @pl.loop(0, C)
    def _(r):
      for j in range(HH // 16):
        br0[pl.ds(r, 1), pl.ds(j * 16, 16)] = zv

    for kk in range(ROWIT):
      g = kk * NSUB + s

      @pl.when(g < NROWCH)
      def _():
        pltpu.sync_copy(br0, acc.at[pl.ds(g * C, C)])

    plsc.subcore_barrier()

    def idx_start(q, off):
      pltpu.make_async_copy(src_hbm.at[pl.ds(off, C)], sidx[q], semi[q]).start()
      pltpu.make_async_copy(dst_hbm.at[pl.ds(off, C)], didx[q], semi[q]).start()

    def idx_wait(q):
      pltpu.make_async_copy(src_hbm.at[pl.ds(0, C)], sidx[q], semi[q]).wait()
      pltpu.make_async_copy(dst_hbm.at[pl.ds(0, C)], didx[q], semi[q]).wait()

    def g_start(b, q, off):
      pltpu.make_async_copy(as_hbm.at[c].at[sidx[q]], sr[b], semg[b]).start()
      pltpu.make_async_copy(ad_hbm.at[c].at[didx[q]], dr[b], semg[b]).start()
      pltpu.make_async_copy(b_hbm.at[c].at[pl.ds(off, C)], br[b],
                            semg[b]).start()

    def g_wait(b):
      pltpu.make_async_copy(as_hbm.at[c].at[sidx[0]], sr[b], semg[b]).wait()
      pltpu.make_async_copy(ad_hbm.at[c].at[didx[0]], dr[b], semg[b]).wait()
      pltpu.make_async_copy(b_hbm.at[c].at[pl.ds(0, C)], br[b],
                            semg[b]).wait()

    def s_start(b, q):
      pltpu.make_async_copy(br[b], acc.at[didx[q]], sems[b]).start(add=True)

    def s_wait(b):
      pltpu.make_async_copy(br[b], acc.at[didx[0]], sems[b]).wait()

    def compute(b):
      @pl.loop(0, C)
      def _(r):
        for j in range(HH // 16):
          slr = (pl.ds(r, 1), pl.ds(j * 16, 16))
          br[b][slr] = jnp.maximum(br[b][slr] + sr[b][slr] + dr[b][slr], 0.0)

    pltpu.sync_copy(src_hbm.at[pl.ds(ebase, C)], si0)
    pltpu.sync_copy(dst_hbm.at[pl.ds(ebase, C)], di0)
    idx_start(1, ebase + C)
    g_start(0, 0, ebase)

    @pl.loop(0, NQ)
    def _(t):
      base = ebase + t * (4 * C)
      for jj in range(4):
        b = jj & 1
        nb = b ^ 1
        qn = (jj + 1) % 4
        qp = (jj + 2) % 4
        g_wait(b)
        if jj == 0:
          @pl.when(t > 0)
          def _(nb=nb):
            s_wait(nb)
        else:
          s_wait(nb)
        if jj < 2:
          idx_start(qp, base + (jj + 2) * C)
        else:
          @pl.when(t < NQ - 1)
          def _(jj=jj, qp=qp):
            idx_start(qp, base + (jj + 2) * C)
        if jj < 3:
          idx_wait(qn)
          g_start(nb, qn, base + (jj + 1) * C)
        else:
          @pl.when(t < NQ - 1)
          def _(jj=jj, qn=qn, nb=nb):
            idx_wait(qn)
            g_start(nb, qn, base + (jj + 1) * C)
        compute(b)
        s_start(b, jj)

    s_wait(1)

    plsc.subcore_barrier()

    for kk in range(ROWIT):
      g = kk * NSUB + s

      @pl.when(g < NROWCH)
      def _():
        off = g * C
        pltpu.sync_copy(acc.at[pl.ds(off, C)], t_hbm.at[c].at[pl.ds(off, C)])

  return k(a_src, a_dst, b_edges, src_idx, dst_idx)


C2 = 80
EPT2 = NE // 2 // NSUB
NCH2 = EPT2 // C2


def _sc_degree(dst_idx):
  mesh = plsc.VectorSubcoreMesh(core_axis_name="c", subcore_axis_name="s")

  @functools.partial(
      pl.kernel,
      mesh=mesh,
      out_type=jax.ShapeDtypeStruct((2, N, HH), _f32),
      scratch_types=[
          pltpu.VMEM((C2,), jnp.int32),
          pltpu.VMEM((C2,), jnp.int32),
          pltpu.VMEM((C2, HH), _f32),
          pltpu.VMEM((C, HH), _f32),
          pltpu.SemaphoreType.DMA,
          pltpu.SemaphoreType.DMA,
          pltpu.SemaphoreType.DMA,
          pltpu.SemaphoreType.DMA,
          pltpu.VMEM_SHARED((N, HH), _f32),
      ],
  )
  def k(dst_hbm, deg_hbm, di0, di1, ones_v, zrows, semi0, semi1,
        sems0, sems1, acc):
    c = lax.axis_index("c")
    s = lax.axis_index("s")
    didx = (di0, di1)
    semi = (semi0, semi1)
    sems = (sems0, sems1)

    zv = jnp.zeros((1, 16), _f32)
    ov = jnp.ones((1, 16), _f32)

    @pl.loop(0, C2)
    def _(r):
      for j in range(HH // 16):
        ones_v[pl.ds(r, 1), pl.ds(j * 16, 16)] = ov

    @pl.loop(0, C)
    def _(r):
      for j in range(HH // 16):
        zrows[pl.ds(r, 1), pl.ds(j * 16, 16)] = zv

    for kk in range(ROWIT):
      g = kk * NSUB + s

      @pl.when(g < NROWCH)
      def _():
        pltpu.sync_copy(zrows, acc.at[pl.ds(g * C, C)])

    plsc.subcore_barrier()

    ebase = (c * NSUB + s) * EPT2

    def idx_start(b, off):
      pltpu.make_async_copy(dst_hbm.at[pl.ds(off, C2)], didx[b],
                            semi[b]).start()

    def idx_wait(b):
      pltpu.make_async_copy(dst_hbm.at[pl.ds(0, C2)], didx[b], semi[b]).wait()

    def s_start(b):
      pltpu.make_async_copy(ones_v, acc.at[didx[b]], sems[b]).start(add=True)

    def s_wait(b):
      pltpu.make_async_copy(ones_v, acc.at[didx[0]], sems[b]).wait()

    pltpu.sync_copy(dst_hbm.at[pl.ds(ebase, C2)], di0)

    @pl.loop(0, NCH2 // 2)
    def _(t):
      base = ebase + t * (2 * C2)

      @pl.when(t > 0)
      def _():
        s_wait(1)

      idx_start(1, base + C2)
      s_start(0)
      s_wait(0)
      idx_wait(1)

      @pl.when(t < NCH2 // 2 - 1)
      def _():
        idx_start(0, base + 2 * C2)

      s_start(1)

      @pl.when(t < NCH2 // 2 - 1)
      def _():
        idx_wait(0)

    s_wait(1)

    plsc.subcore_barrier()

    for kk in range(ROWIT):
      g = kk * NSUB + s

      @pl.when(g < NROWCH)
      def _():
        off = g * C
        pltpu.sync_copy(acc.at[pl.ds(off, C)], deg_hbm.at[c].at[pl.ds(off, C)])

  return k(dst_idx)


def _tc_prep(V, Ws, Wd, eu_W2, Wn1_a, eu_b2r, zr, ur, Wn1_z, Wn1_u, nu_b1r):
  def body(v_r, ws_r, wd_r, w2_r, wa_r, b2_r, z_r, u_r, wz_r, wu_r, nb1_r,
           as_o, ad_o, m_o, crow_o, crow2_o):
    v = v_r[...]
    as_o[0] = jnp.dot(v, ws_r[0], preferred_element_type=_f32)
    as_o[1] = jnp.dot(v, ws_r[1], preferred_element_type=_f32)
    ad_o[0] = jnp.dot(v, wd_r[0], preferred_element_type=_f32)
    ad_o[1] = jnp.dot(v, wd_r[1], preferred_element_type=_f32)
    m_o[...] = jnp.dot(w2_r[...], wa_r[...], preferred_element_type=_f32)
    crow_o[...] = (jnp.dot(z_r[...], wz_r[...], preferred_element_type=_f32)
                   + jnp.dot(u_r[...], wu_r[...], preferred_element_type=_f32)
                   + nb1_r[...])
    crow2_o[...] = jnp.dot(b2_r[...], wa_r[...], preferred_element_type=_f32)

  return pl.pallas_call(
      body,
      out_shape=[jax.ShapeDtypeStruct((2, N, HH), _f32),
                 jax.ShapeDtypeStruct((2, N, HH), _f32),
                 jax.ShapeDtypeStruct((H, H), _f32),
                 jax.ShapeDtypeStruct((1, H), _f32),
                 jax.ShapeDtypeStruct((1, H), _f32)],
  )(V, Ws, Wd, eu_W2, Wn1_a, eu_b2r, zr, ur, Wn1_z, Wn1_u, nu_b1r)


_BBLK = 16000


def _tc_edge_bias(E, WE, b1r):
  def body(e_r, we_r, b1_r, b_o):
    e = e_r[...]
    b_o[0] = jnp.dot(e, we_r[0], preferred_element_type=_f32) + b1_r[0]
    b_o[1] = jnp.dot(e, we_r[1], preferred_element_type=_f32) + b1_r[1]

  nblk = NE // _BBLK
  return pl.pallas_call(
      body,
      grid=(nblk,),
      in_specs=[
          pl.BlockSpec((_BBLK, 16), lambda i: (i, 0)),
          pl.BlockSpec((2, 16, HH), lambda i: (0, 0, 0)),
          pl.BlockSpec((2, 1, HH), lambda i: (0, 0, 0)),
      ],
      out_specs=pl.BlockSpec((2, _BBLK, HH), lambda i: (0, i, 0)),
      out_shape=jax.ShapeDtypeStruct((2, NE, HH), _f32),
  )(E, WE, b1r)


_NBLK = 1000


def _tc_node(Tflat, Degs, V, M, Wn1_v, crow, crow2, nu_W2, nu_b2r):
  def body(t0_r, t1_r, dg0_r, dg1_r, v_r, m_r, wv_r, crow_r, crow2_r,
           w2_r, b2_r, vp_o, vsum_o, vmax_o):
    i = pl.program_id(0)
    degraw = dg0_r[:, 0:1] + dg1_r[:, 0:1]
    deg = jnp.maximum(degraw, 1.0)
    ind = (degraw > 0.0).astype(_f32)
    x0 = t0_r[...] / deg
    x1 = t1_r[...] / deg
    pre = (jnp.dot(x0, m_r[0:HH], preferred_element_type=_f32)
           + jnp.dot(x1, m_r[HH:], preferred_element_type=_f32)
           + jnp.dot(v_r[...], wv_r[...], preferred_element_type=_f32)
           + crow_r[...] + ind * crow2_r[...])
    h = jnp.maximum(pre, 0.0)
    out = jnp.dot(h, w2_r[...], preferred_element_type=_f32) + b2_r[...]
    vp_o[...] = out
    psum = jnp.sum(out, axis=0, keepdims=True)
    pmax = jnp.max(out, axis=0, keepdims=True)

    @pl.when(i == 0)
    def _():
      vsum_o[...] = psum
      vmax_o[...] = pmax

    @pl.when(i > 0)
    def _():
      vsum_o[...] = vsum_o[...] + psum
      vmax_o[...] = jnp.maximum(vmax_o[...], pmax)

  full2 = lambda shape: pl.BlockSpec(shape, lambda i: (0, 0))
  return pl.pallas_call(
      body,
      grid=(N // _NBLK,),
      in_specs=[
          pl.BlockSpec((_NBLK, HH), lambda i: (i, 0)),
          pl.BlockSpec((_NBLK, HH), lambda i: (N // _NBLK + i, 0)),
          pl.BlockSpec((_NBLK, HH), lambda i: (i, 0)),
          pl.BlockSpec((_NBLK, HH), lambda i: (N // _NBLK + i, 0)),
          pl.BlockSpec((_NBLK, D), lambda i: (i, 0)),
          full2((H, H)),
          full2((D, H)),
          full2((1, H)),
          full2((1, H)),
          full2((H, H)),
          full2((1, H)),
      ],
      out_specs=[
          pl.BlockSpec((_NBLK, H), lambda i: (i, 0)),
          full2((1, H)),
          full2((1, H)),
      ],
      out_shape=[
          jax.ShapeDtypeStruct((N, H), _f32),
          jax.ShapeDtypeStruct((1, H), _f32),
          jax.ShapeDtypeStruct((1, H), _f32),
      ],
  )(Tflat, Tflat, Degs, Degs, V, M, Wn1_v, crow, crow2, nu_W2, nu_b2r)


def _tc_global(vsum, vmax, zr, ur, gn_W1, gn_b1r, gn_W2, gn_b2r,
               gu_W1, gu_b1r, gu_W2, gu_b2r):
  def body(vs_r, vm_r, z_r, u_r, w1_r, b1_r, w2_r, b2_r,
           uw1_r, ub1_r, uw2_r, ub2_r, zp_o, up_o):
    vmean = vs_r[...] * (1.0 / N)
    h = jnp.maximum(
        jnp.dot(vmean, w1_r[0:H], preferred_element_type=_f32)
        + jnp.dot(z_r[...], w1_r[H:], preferred_element_type=_f32)
        + b1_r[...], 0.0)
    zp = jnp.dot(h, w2_r[...], preferred_element_type=_f32) + b2_r[...]
    zp_o[...] = zp
    h2 = jnp.maximum(
        jnp.dot(vmean, uw1_r[0:H], preferred_element_type=_f32)
        + jnp.dot(vm_r[...], uw1_r[H:2 * H], preferred_element_type=_f32)
        + jnp.dot(zp, uw1_r[2 * H:3 * H], preferred_element_type=_f32)
        + jnp.dot(u_r[...], uw1_r[3 * H:], preferred_element_type=_f32)
        + ub1_r[...], 0.0)
    up_o[...] = jnp.dot(h2, uw2_r[...], preferred_element_type=_f32) + ub2_r[...]

  return pl.pallas_call(
      body,
      out_shape=[
          jax.ShapeDtypeStruct((1, H), _f32),
          jax.ShapeDtypeStruct((1, D), _f32),
      ],
  )(vsum, vmax, zr, ur, gn_W1, gn_b1r, gn_W2, gn_b2r,
    gu_W1, gu_b1r, gu_W2, gu_b2r)


def kernel(V, E, edge_index, u, z,
           eu_W1, eu_b1, eu_W2, eu_b2,
           nu_W1, nu_b1, nu_W2, nu_b2,
           gn_W1, gn_b1, gn_W2, gn_b2,
           gu_W1, gu_b1, gu_W2, gu_b2):
  src = edge_index[0].astype(jnp.int32)
  dst = edge_index[1].astype(jnp.int32)

  WE = eu_W1[:16].reshape(16, 2, HH).transpose(1, 0, 2)
  Ws = eu_W1[16:16 + D].reshape(D, 2, HH).transpose(1, 0, 2)
  Wd = eu_W1[16 + D:].reshape(D, 2, HH).transpose(1, 0, 2)
  b1r = eu_b1.reshape(2, 1, HH)

  Wn1_a = nu_W1[0:H]
  Wn1_v = nu_W1[H:H + D]
  Wn1_z = nu_W1[H + D:2 * H + D]
  Wn1_u = nu_W1[2 * H + D:]

  zr = z.reshape(1, H)
  ur = u.reshape(1, D)

  As, Ad, M, crow, crow2 = _tc_prep(
      V, Ws, Wd, eu_W2, Wn1_a, eu_b2.reshape(1, H), zr, ur,
      Wn1_z, Wn1_u, nu_b1.reshape(1, H))
  B = _tc_edge_bias(E, WE, b1r)
  Degs = _sc_degree(dst).reshape(2 * N, HH)
  T = _sc_edge_aggregate(As, Ad, B, src, dst).reshape(2 * N, HH)
  V_prime, vsum, vmax = _tc_node(T, Degs, V, M, Wn1_v, crow, crow2,
                                 nu_W2, nu_b2.reshape(1, H))
  zp, up = _tc_global(vsum, vmax, zr, ur,
                      gn_W1, gn_b1.reshape(1, H), gn_W2, gn_b2.reshape(1, H),
                      gu_W1, gu_b1.reshape(1, H), gu_W2, gu_b2.reshape(1, D))
  return (V_prime, up.reshape(D), zp.reshape(H))


# speedup vs baseline: 2.8455x; 1.0030x over previous
import functools

import jax
import jax.numpy as jnp
from jax import lax
from jax.experimental import pallas as pl
from jax.experimental.pallas import tpu as pltpu
from jax.experimental.pallas import tpu_sc as plsc

N = 10000
NE = 320000
D = 128
H = 256
HH = 128
NSUB = 16
EPT = NE // NSUB
C = 40
NCH = EPT // C
NQ = NCH // 4
NROWCH = N // C
ROWIT = -(-NROWCH // NSUB)

_f32 = jnp.float32


def _sc_edge_aggregate(a_src, a_dst, b_edges, src_idx, dst_idx):
  mesh = plsc.VectorSubcoreMesh(core_axis_name="c", subcore_axis_name="s")

  @functools.partial(
      pl.kernel,
      mesh=mesh,
      out_type=jax.ShapeDtypeStruct((2, N, HH), _f32),
      scratch_types=[
          pltpu.VMEM((C,), jnp.int32),
          pltpu.VMEM((C,), jnp.int32),
          pltpu.VMEM((C,), jnp.int32),
          pltpu.VMEM((C,), jnp.int32),
          pltpu.VMEM((C,), jnp.int32),
          pltpu.VMEM((C,), jnp.int32),
          pltpu.VMEM((C,), jnp.int32),
          pltpu.VMEM((C,), jnp.int32),
          pltpu.VMEM((C, HH), _f32),
          pltpu.VMEM((C, HH), _f32),
          pltpu.VMEM((C, HH), _f32),
          pltpu.VMEM((C, HH), _f32),
          pltpu.VMEM((C, HH), _f32),
          pltpu.VMEM((C, HH), _f32),
          pltpu.SemaphoreType.DMA,
          pltpu.SemaphoreType.DMA,
          pltpu.SemaphoreType.DMA,
          pltpu.SemaphoreType.DMA,
          pltpu.SemaphoreType.DMA,
          pltpu.SemaphoreType.DMA,
          pltpu.SemaphoreType.DMA,
          pltpu.SemaphoreType.DMA,
          pltpu.VMEM_SHARED((N, HH), _f32),
      ],
  )
  def k(as_hbm, ad_hbm, b_hbm, src_hbm, dst_hbm, t_hbm,
        si0, si1, si2, si3, di0, di1, di2, di3,
        sr0, sr1, dr0, dr1, br0, br1,
        semg0, semg1, semi0, semi1, semi2, semi3, sems0, sems1, acc):
    c = lax.axis_index("c")
    s = lax.axis_index("s")
    sidx = (si0, si1, si2, si3)
    didx = (di0, di1, di2, di3)
    sr = (sr0, sr1)
    dr = (dr0, dr1)
    br = (br0, br1)
    semg = (semg0, semg1)
    semi = (semi0, semi1, semi2, semi3)
    sems = (sems0, sems1)
    ebase = s * EPT

    zv = jnp.zeros((1, 16), _f32)

    @pl.loop(0, C)
    def _(r):
      for j in range(HH // 16):
        br0[pl.ds(r, 1), pl.ds(j * 16, 16)] = zv

    for kk in range(ROWIT):
      g = kk * NSUB + s

      @pl.when(g < NROWCH)
      def _():
        pltpu.sync_copy(br0, acc.at[pl.ds(g * C, C)])

    plsc.subcore_barrier()

    def idx_start(q, off):
      pltpu.make_async_copy(src_hbm.at[pl.ds(off, C)], sidx[q], semi[q]).start()
      pltpu.make_async_copy(dst_hbm.at[pl.ds(off, C)], didx[q], semi[q]).start()

    def idx_wait(q):
      pltpu.make_async_copy(src_hbm.at[pl.ds(0, C)], sidx[q], semi[q]).wait()
      pltpu.make_async_copy(dst_hbm.at[pl.ds(0, C)], didx[q], semi[q]).wait()

    def g_start(b, q, off):
      pltpu.make_async_copy(as_hbm.at[c].at[sidx[q]], sr[b], semg[b]).start()
      pltpu.make_async_copy(ad_hbm.at[c].at[didx[q]], dr[b], semg[b]).start()
      pltpu.make_async_copy(b_hbm.at[c].at[pl.ds(off, C)], br[b],
                            semg[b]).start()

    def g_wait(b):
      pltpu.make_async_copy(as_hbm.at[c].at[sidx[0]], sr[b], semg[b]).wait()
      pltpu.make_async_copy(ad_hbm.at[c].at[didx[0]], dr[b], semg[b]).wait()
      pltpu.make_async_copy(b_hbm.at[c].at[pl.ds(0, C)], br[b],
                            semg[b]).wait()

    def s_start(b, q):
      pltpu.make_async_copy(br[b], acc.at[didx[q]], sems[b]).start(add=True)

    def s_wait(b):
      pltpu.make_async_copy(br[b], acc.at[didx[0]], sems[b]).wait()

    def compute(b):
      @pl.loop(0, C)
      def _(r):
        for j in range(HH // 16):
          slr = (pl.ds(r, 1), pl.ds(j * 16, 16))
          br[b][slr] = jnp.maximum(br[b][slr] + sr[b][slr] + dr[b][slr], 0.0)

    pltpu.sync_copy(src_hbm.at[pl.ds(ebase, C)], si0)
    pltpu.sync_copy(dst_hbm.at[pl.ds(ebase, C)], di0)
    idx_start(1, ebase + C)
    g_start(0, 0, ebase)

    @pl.loop(0, NQ)
    def _(t):
      base = ebase + t * (4 * C)
      for jj in range(4):
        b = jj & 1
        nb = b ^ 1
        qn = (jj + 1) % 4
        qp = (jj + 2) % 4
        g_wait(b)
        if jj == 0:
          @pl.when(t > 0)
          def _(nb=nb):
            s_wait(nb)
        else:
          s_wait(nb)
        if jj < 2:
          idx_start(qp, base + (jj + 2) * C)
        else:
          @pl.when(t < NQ - 1)
          def _(jj=jj, qp=qp):
            idx_start(qp, base + (jj + 2) * C)
        if jj < 3:
          idx_wait(qn)
          g_start(nb, qn, base + (jj + 1) * C)
        else:
          @pl.when(t < NQ - 1)
          def _(jj=jj, qn=qn, nb=nb):
            idx_wait(qn)
            g_start(nb, qn, base + (jj + 1) * C)
        compute(b)
        s_start(b, jj)

    s_wait(1)

    plsc.subcore_barrier()

    for kk in range(ROWIT):
      g = kk * NSUB + s

      @pl.when(g < NROWCH)
      def _():
        off = g * C
        pltpu.sync_copy(acc.at[pl.ds(off, C)], t_hbm.at[c].at[pl.ds(off, C)])

  return k(a_src, a_dst, b_edges, src_idx, dst_idx)


C2 = 80
EPT2 = NE // 2 // NSUB
NCH2 = EPT2 // C2


def _sc_degree(dst_idx):
  mesh = plsc.VectorSubcoreMesh(core_axis_name="c", subcore_axis_name="s")

  @functools.partial(
      pl.kernel,
      mesh=mesh,
      out_type=jax.ShapeDtypeStruct((2, N, HH), _f32),
      scratch_types=[
          pltpu.VMEM((C2,), jnp.int32),
          pltpu.VMEM((C2,), jnp.int32),
          pltpu.VMEM((C2, HH), _f32),
          pltpu.VMEM((C, HH), _f32),
          pltpu.SemaphoreType.DMA,
          pltpu.SemaphoreType.DMA,
          pltpu.SemaphoreType.DMA,
          pltpu.SemaphoreType.DMA,
          pltpu.VMEM_SHARED((N, HH), _f32),
      ],
  )
  def k(dst_hbm, deg_hbm, di0, di1, ones_v, zrows, semi0, semi1,
        sems0, sems1, acc):
    c = lax.axis_index("c")
    s = lax.axis_index("s")
    didx = (di0, di1)
    semi = (semi0, semi1)
    sems = (sems0, sems1)

    zv = jnp.zeros((1, 16), _f32)
    ov = jnp.ones((1, 16), _f32)

    @pl.loop(0, C2)
    def _(r):
      for j in range(HH // 16):
        ones_v[pl.ds(r, 1), pl.ds(j * 16, 16)] = ov

    @pl.loop(0, C)
    def _(r):
      for j in range(HH // 16):
        zrows[pl.ds(r, 1), pl.ds(j * 16, 16)] = zv

    for kk in range(ROWIT):
      g = kk * NSUB + s

      @pl.when(g < NROWCH)
      def _():
        pltpu.sync_copy(zrows, acc.at[pl.ds(g * C, C)])

    plsc.subcore_barrier()

    ebase = (c * NSUB + s) * EPT2

    def idx_start(b, off):
      pltpu.make_async_copy(dst_hbm.at[pl.ds(off, C2)], didx[b],
                            semi[b]).start()

    def idx_wait(b):
      pltpu.make_async_copy(dst_hbm.at[pl.ds(0, C2)], didx[b], semi[b]).wait()

    def s_start(b):
      pltpu.make_async_copy(ones_v, acc.at[didx[b]], sems[b]).start(add=True)

    def s_wait(b):
      pltpu.make_async_copy(ones_v, acc.at[didx[0]], sems[b]).wait()

    pltpu.sync_copy(dst_hbm.at[pl.ds(ebase, C2)], di0)

    @pl.loop(0, NCH2 // 2)
    def _(t):
      base = ebase + t * (2 * C2)

      @pl.when(t > 0)
      def _():
        s_wait(1)

      idx_start(1, base + C2)
      s_start(0)
      s_wait(0)
      idx_wait(1)

      @pl.when(t < NCH2 // 2 - 1)
      def _():
        idx_start(0, base + 2 * C2)

      s_start(1)

      @pl.when(t < NCH2 // 2 - 1)
      def _():
        idx_wait(0)

    s_wait(1)

    plsc.subcore_barrier()

    for kk in range(ROWIT):
      g = kk * NSUB + s

      @pl.when(g < NROWCH)
      def _():
        off = g * C
        pltpu.sync_copy(acc.at[pl.ds(off, C)], deg_hbm.at[c].at[pl.ds(off, C)])

  return k(dst_idx)


def _tc_prep(V, Ws, Wd, eu_W2, Wn1_a, eu_b2r, zr, ur, Wn1_z, Wn1_u, nu_b1r):
  def body(v_r, ws_r, wd_r, w2_r, wa_r, b2_r, z_r, u_r, wz_r, wu_r, nb1_r,
           as_o, ad_o, m_o, crow_o, crow2_o):
    v = v_r[...]
    as_o[0] = jnp.dot(v, ws_r[0], preferred_element_type=_f32)
    as_o[1] = jnp.dot(v, ws_r[1], preferred_element_type=_f32)
    ad_o[0] = jnp.dot(v, wd_r[0], preferred_element_type=_f32)
    ad_o[1] = jnp.dot(v, wd_r[1], preferred_element_type=_f32)
    m_o[...] = jnp.dot(w2_r[...], wa_r[...], preferred_element_type=_f32)
    crow_o[...] = (jnp.dot(z_r[...], wz_r[...], preferred_element_type=_f32)
                   + jnp.dot(u_r[...], wu_r[...], preferred_element_type=_f32)
                   + nb1_r[...])
    crow2_o[...] = jnp.dot(b2_r[...], wa_r[...], preferred_element_type=_f32)

  return pl.pallas_call(
      body,
      out_shape=[jax.ShapeDtypeStruct((2, N, HH), _f32),
                 jax.ShapeDtypeStruct((2, N, HH), _f32),
                 jax.ShapeDtypeStruct((H, H), _f32),
                 jax.ShapeDtypeStruct((1, H), _f32),
                 jax.ShapeDtypeStruct((1, H), _f32)],
  )(V, Ws, Wd, eu_W2, Wn1_a, eu_b2r, zr, ur, Wn1_z, Wn1_u, nu_b1r)


_BBLK = 16000


def _tc_edge_bias(E, WE, b1r):
  def body(e_r, we_r, b1_r, b_o):
    e = e_r[...]
    b_o[0] = jnp.dot(e, we_r[0], preferred_element_type=_f32) + b1_r[0]
    b_o[1] = jnp.dot(e, we_r[1], preferred_element_type=_f32) + b1_r[1]

  nblk = NE // _BBLK
  return pl.pallas_call(
      body,
      grid=(nblk,),
      in_specs=[
          pl.BlockSpec((_BBLK, 16), lambda i: (i, 0)),
          pl.BlockSpec((2, 16, HH), lambda i: (0, 0, 0)),
          pl.BlockSpec((2, 1, HH), lambda i: (0, 0, 0)),
      ],
      out_specs=pl.BlockSpec((2, _BBLK, HH), lambda i: (0, i, 0)),
      out_shape=jax.ShapeDtypeStruct((2, NE, HH), _f32),
  )(E, WE, b1r)


_NBLK = 1000


def _tc_node(Tflat, Degs, V, M, Wn1_v, crow, crow2, nu_W2, nu_b2r,
             zr, ur, gn_W1, gn_b1r, gn_W2, gn_b2r,
             gu_W1, gu_b1r, gu_W2, gu_b2r):
  def body(t0_r, t1_r, dg0_r, dg1_r, v_r, m_r, wv_r, crow_r, crow2_r,
           w2_r, b2_r, z_r, u_r, w1_r, b1_r, w2g_r, b2g_r,
           uw1_r, ub1_r, uw2_r, ub2_r, vp_o, vsum_o, vmax_o, zp_o, up_o):
    i = pl.program_id(0)
    degraw = dg0_r[:, 0:1] + dg1_r[:, 0:1]
    deg = jnp.maximum(degraw, 1.0)
    ind = (degraw > 0.0).astype(_f32)
    x0 = t0_r[...] / deg
    x1 = t1_r[...] / deg
    pre = (jnp.dot(x0, m_r[0:HH], preferred_element_type=_f32)
           + jnp.dot(x1, m_r[HH:], preferred_element_type=_f32)
           + jnp.dot(v_r[...], wv_r[...], preferred_element_type=_f32)
           + crow_r[...] + ind * crow2_r[...])
    h = jnp.maximum(pre, 0.0)
    out = jnp.dot(h, w2_r[...], preferred_element_type=_f32) + b2_r[...]
    vp_o[...] = out
    psum = jnp.sum(out, axis=0, keepdims=True)
    pmax = jnp.max(out, axis=0, keepdims=True)

    @pl.when(i == 0)
    def _():
      vsum_o[...] = psum
      vmax_o[...] = pmax

    @pl.when(i > 0)
    def _():
      vsum_o[...] = vsum_o[...] + psum
      vmax_o[...] = jnp.maximum(vmax_o[...], pmax)

    @pl.when(i == N // _NBLK - 1)
    def _():
      vmean = vsum_o[...] * (1.0 / N)
      vmx = vmax_o[...]
      hz = jnp.maximum(
          jnp.dot(vmean, w1_r[0:H], preferred_element_type=_f32)
          + jnp.dot(z_r[...], w1_r[H:], preferred_element_type=_f32)
          + b1_r[...], 0.0)
      zp = jnp.dot(hz, w2g_r[...], preferred_element_type=_f32) + b2g_r[...]
      zp_o[...] = zp
      hu = jnp.maximum(
          jnp.dot(vmean, uw1_r[0:H], preferred_element_type=_f32)
          + jnp.dot(vmx, uw1_r[H:2 * H], preferred_element_type=_f32)
          + jnp.dot(zp, uw1_r[2 * H:3 * H], preferred_element_type=_f32)
          + jnp.dot(u_r[...], uw1_r[3 * H:], preferred_element_type=_f32)
          + ub1_r[...], 0.0)
      up_o[...] = (jnp.dot(hu, uw2_r[...], preferred_element_type=_f32)
                   + ub2_r[...])

  full2 = lambda shape: pl.BlockSpec(shape, lambda i: (0, 0))
  full3 = lambda shape: pl.BlockSpec(shape, lambda i: (0, 0))
  return pl.pallas_call(
      body,
      grid=(N // _NBLK,),
      in_specs=[
          pl.BlockSpec((_NBLK, HH), lambda i: (i, 0)),
          pl.BlockSpec((_NBLK, HH), lambda i: (N // _NBLK + i, 0)),
          pl.BlockSpec((_NBLK, HH), lambda i: (i, 0)),
          pl.BlockSpec((_NBLK, HH), lambda i: (N // _NBLK + i, 0)),
          pl.BlockSpec((_NBLK, D), lambda i: (i, 0)),
          full2((H, H)),
          full2((D, H)),
          full2((1, H)),
          full2((1, H)),
          full2((H, H)),
          full2((1, H)),
          full2((1, H)),
          full2((1, D)),
          full2((2 * H, H)),
          full2((1, H)),
          full2((H, H)),
          full2((1, H)),
          full2((3 * H + D, H)),
          full2((1, H)),
          full2((H, D)),
          full2((1, D)),
      ],
      out_specs=[
          pl.BlockSpec((_NBLK, H), lambda i: (i, 0)),
          full2((1, H)),
          full2((1, H)),
          full2((1, H)),
          full2((1, D)),
      ],
      out_shape=[
          jax.ShapeDtypeStruct((N, H), _f32),
          jax.ShapeDtypeStruct((1, H), _f32),
          jax.ShapeDtypeStruct((1, H), _f32),
          jax.ShapeDtypeStruct((1, H), _f32),
          jax.ShapeDtypeStruct((1, D), _f32),
      ],
  )(Tflat, Tflat, Degs, Degs, V, M, Wn1_v, crow, crow2, nu_W2, nu_b2r,
    zr, ur, gn_W1, gn_b1r, gn_W2, gn_b2r, gu_W1, gu_b1r, gu_W2, gu_b2r)


def kernel(V, E, edge_index, u, z,
           eu_W1, eu_b1, eu_W2, eu_b2,
           nu_W1, nu_b1, nu_W2, nu_b2,
           gn_W1, gn_b1, gn_W2, gn_b2,
           gu_W1, gu_b1, gu_W2, gu_b2):
  src = edge_index[0].astype(jnp.int32)
  dst = edge_index[1].astype(jnp.int32)

  WE = eu_W1[:16].reshape(16, 2, HH).transpose(1, 0, 2)
  Ws = eu_W1[16:16 + D].reshape(D, 2, HH).transpose(1, 0, 2)
  Wd = eu_W1[16 + D:].reshape(D, 2, HH).transpose(1, 0, 2)
  b1r = eu_b1.reshape(2, 1, HH)

  Wn1_a = nu_W1[0:H]
  Wn1_v = nu_W1[H:H + D]
  Wn1_z = nu_W1[H + D:2 * H + D]
  Wn1_u = nu_W1[2 * H + D:]

  zr = z.reshape(1, H)
  ur = u.reshape(1, D)

  As, Ad, M, crow, crow2 = _tc_prep(
      V, Ws, Wd, eu_W2, Wn1_a, eu_b2.reshape(1, H), zr, ur,
      Wn1_z, Wn1_u, nu_b1.reshape(1, H))
  B = _tc_edge_bias(E, WE, b1r)
  Degs = _sc_degree(dst).reshape(2 * N, HH)
  T = _sc_edge_aggregate(As, Ad, B, src, dst).reshape(2 * N, HH)
  V_prime, _, _, zp, up = _tc_node(
      T, Degs, V, M, Wn1_v, crow, crow2, nu_W2, nu_b2.reshape(1, H),
      zr, ur, gn_W1, gn_b1.reshape(1, H), gn_W2, gn_b2.reshape(1, H),
      gu_W1, gu_b1.reshape(1, H), gu_W2, gu_b2.reshape(1, D))
  return (V_prime, up.reshape(D), zp.reshape(H))
